# baseline (speedup 1.0000x reference)
import jax
import jax.numpy as jnp
from jax import lax
from jax.experimental import pallas as pl
from jax.experimental.pallas import tpu as pltpu
from jax.experimental.pallas import tpu_sc as plsc

N = 10000
D = 128
E = 320000

NC = 2
NS = 16
NW = NC * NS
EPW = E // NW
CH = 50
NCH = EPW // CH
NCHP = NCH + 2
NP = 10240
RPS = NP // NS
CNTW = 16

_f32 = jnp.float32


def _build_segsum(with_count):
  mesh = plsc.VectorSubcoreMesh(
      core_axis_name="c", subcore_axis_name="s",
      num_cores=NC, num_subcores=NS)

  out_type = jax.ShapeDtypeStruct((NC, NP, D), _f32)
  scratch = [
      pltpu.MemorySpace.VMEM((NCHP, CH), jnp.int32),
      pltpu.MemorySpace.VMEM((NCHP, CH), jnp.int32),
      pltpu.MemorySpace.VMEM((CH, D), _f32),
      pltpu.MemorySpace.VMEM((CH, D), _f32),
      pltpu.MemorySpace.VMEM((CH, D), _f32),
      pltpu.MemorySpace.VMEM((CH, D), _f32),
      pltpu.MemorySpace.VMEM_SHARED((NP, D), _f32),
      pltpu.SemaphoreType.DMA,
      pltpu.SemaphoreType.DMA,
      pltpu.SemaphoreType.DMA,
      pltpu.SemaphoreType.DMA,
  ]

  def body(y, src, dst, zrow, acc_out,
           src_v, dst_v, rows0, rows1, rows2, rows3, acc_sh,
           sem0, sem1, sem2, sem3):
    c = lax.axis_index("c")
    s = lax.axis_index("s")
    wid = s * NC + c
    rows = (rows0, rows1, rows2, rows3)
    sems = (sem0, sem1, sem2, sem3)

    def gather(j, b):
      pltpu.async_copy(y.at[src_v.at[j]], rows[b], sems[b])

    def gwait(j, b):
      pltpu.make_async_copy(y.at[src_v.at[j]], rows[b], sems[b]).wait()

    pltpu.sync_copy(src.at[wid], src_v)
    pltpu.sync_copy(dst.at[wid], dst_v)
    gather(0, 0)
    gather(1, 1)
    pltpu.sync_copy(zrow, acc_sh.at[pl.ds(s * RPS, RPS)])
    plsc.subcore_barrier()

    def step(g, carry):
      j = 4 * g
      for b in range(4):
        jj = j + b
        gwait(jj, b)
        pltpu.sync_copy(rows[b], acc_sh.at[dst_v.at[jj]], add=True)
        gather(jj + 2, (b + 2) % 4)
      return carry

    lax.fori_loop(0, NCH // 4, step, 0)
    gwait(NCH, 0)
    gwait(NCH + 1, 1)

    plsc.subcore_barrier()
    pltpu.sync_copy(acc_sh.at[pl.ds(s * RPS, RPS)],
                    acc_out.at[c, pl.ds(s * RPS, RPS)])

  return pl.kernel(
      body, out_type=out_type, mesh=mesh, scratch_types=scratch,
      compiler_params=pltpu.CompilerParams(use_tc_tiling_on_sc=False))


def _build_counts():
  mesh = plsc.VectorSubcoreMesh(
      core_axis_name="c", subcore_axis_name="s",
      num_cores=NC, num_subcores=NS)
  out_type = [jax.ShapeDtypeStruct((NC, NP, CNTW), _f32),
              jax.ShapeDtypeStruct((NC, NP, CNTW), _f32)]
  scratch = [
      pltpu.MemorySpace.VMEM((NCHP, CH), jnp.int32),
      pltpu.MemorySpace.VMEM((CH, CNTW), _f32),
      pltpu.MemorySpace.VMEM_SHARED((NP, CNTW), _f32),
      pltpu.SemaphoreType.DMA,
  ]

  def body(dst1, dst2, zcnt, ones, cnt1_out, cnt2_out,
           dst_v, ones_v, cnt_sh, csem):
    c = lax.axis_index("c")
    s = lax.axis_index("s")
    wid = s * NC + c
    K = 8

    pltpu.sync_copy(ones, ones_v)
    for dst, cnt_out in ((dst1, cnt1_out), (dst2, cnt2_out)):
      pltpu.sync_copy(dst.at[wid], dst_v)
      pltpu.sync_copy(zcnt, cnt_sh.at[pl.ds(s * RPS, RPS)])
      plsc.subcore_barrier()

      def step(j, carry):
        pltpu.sync_copy(ones_v, cnt_sh.at[dst_v.at[j]], add=True)
        return carry

      lax.fori_loop(0, NCH, step, 0)
      plsc.subcore_barrier()
      pltpu.sync_copy(cnt_sh.at[pl.ds(s * RPS, RPS)],
                      cnt_out.at[c, pl.ds(s * RPS, RPS)])
      plsc.subcore_barrier()

  return pl.kernel(
      body, out_type=out_type, mesh=mesh, scratch_types=scratch,
      compiler_params=pltpu.CompilerParams(use_tc_tiling_on_sc=False))


_segsum = _build_segsum(True)
_counts = _build_counts()


R = 1000
G = N // R

_row_spec = pl.BlockSpec((R, D), lambda i: (i, 0))
_acc_spec = pl.BlockSpec((NC, R, D), lambda i: (0, i, 0))
_cnt_spec = pl.BlockSpec((NC, R, CNTW), lambda i: (0, i, 0))
_w_spec = pl.BlockSpec((D, D), lambda i: (0, 0))
_b_spec = pl.BlockSpec((1, D), lambda i: (0, 0))


def _tc_y1_body(x_ref, wlt_ref, y_ref):
  y_ref[...] = jnp.dot(x_ref[...], wlt_ref[...], preferred_element_type=_f32)


def _tc_z_body(h_ref, wrt_ref, bl_ref, z_ref):
  z_ref[...] = jnp.dot(h_ref[...], wrt_ref[...],
                       preferred_element_type=_f32) + bl_ref[...]


def _tc_comb_body(acc_ref, cnt_ref, z_ref, m_ref, wlt_ref, y_ref, h_ref):
  agg = acc_ref[0] + acc_ref[1]
  cnt = cnt_ref[0, :, 0:1] + cnt_ref[1, :, 0:1]
  inv = 1.0 / jnp.maximum(cnt, 1.0)
  h = jnp.maximum(z_ref[...] + agg * inv, 0.0) * m_ref[...]
  h_ref[...] = h
  y_ref[...] = jnp.dot(h, wlt_ref[...], preferred_element_type=_f32)


def _tc_final_body(acc_ref, cnt_ref, z_ref, out_ref):
  agg = acc_ref[0] + acc_ref[1]
  cnt = cnt_ref[0, :, 0:1] + cnt_ref[1, :, 0:1]
  inv = 1.0 / jnp.maximum(cnt, 1.0)
  out_ref[...] = z_ref[...] + agg * inv


_nd = jax.ShapeDtypeStruct((N, D), _f32)

_tc_y1 = pl.pallas_call(
    _tc_y1_body, grid=(G,),
    in_specs=[_row_spec, _w_spec],
    out_specs=_row_spec,
    out_shape=_nd)

_tc_z = pl.pallas_call(
    _tc_z_body, grid=(G,),
    in_specs=[_row_spec, _w_spec, _b_spec],
    out_specs=_row_spec,
    out_shape=_nd)

_tc_comb = pl.pallas_call(
    _tc_comb_body, grid=(G,),
    in_specs=[_acc_spec, _cnt_spec, _row_spec, _row_spec, _w_spec],
    out_specs=[_row_spec, _row_spec],
    out_shape=[_nd, _nd])

_tc_final = pl.pallas_call(
    _tc_final_body, grid=(G,),
    in_specs=[_acc_spec, _cnt_spec, _row_spec],
    out_specs=_row_spec,
    out_shape=_nd)


def kernel(x, edge_index, edge_idx_1_1, Wl1, bl1, Wr1, Wl2, bl2, Wr2,
           Wl3, bl3, Wr3):
  def _stage(v):
    v = v.reshape(NW, NCH, CH)
    return jnp.concatenate([v, v[:, :2]], axis=1)

  src1 = _stage(edge_index[0])
  dst1 = _stage(edge_index[1])
  src2 = _stage(edge_idx_1_1[0])
  dst2 = _stage(edge_idx_1_1[1])

  m1 = jax.random.bernoulli(jax.random.key(1), 0.5, (N, D)).astype(_f32) * 2.0
  m2 = jax.random.bernoulli(jax.random.key(2), 0.5, (N, D)).astype(_f32) * 2.0

  zrow = jnp.zeros((RPS, D), _f32)
  zcnt = jnp.zeros((RPS, CNTW), _f32)
  ones = jnp.ones((CH, CNTW), _f32)

  cnt1, cnt2 = _counts(dst1, dst2, zcnt, ones)
  y1 = _tc_y1(x, Wl1.T)
  acc1 = _segsum(y1, src1, dst1, zrow)
  z1 = _tc_z(x, Wr1.T, bl1.reshape(1, D))
  y2, h2 = _tc_comb(acc1, cnt1, z1, m1, Wl2.T)
  acc2 = _segsum(y2, src2, dst2, zrow)
  z2 = _tc_z(h2, Wr2.T, bl2.reshape(1, D))
  y3, h3 = _tc_comb(acc2, cnt2, z2, m2, Wl3.T)
  acc3 = _segsum(y3, src1, dst1, zrow)
  z3 = _tc_z(h3, Wr3.T, bl3.reshape(1, D))
  return _tc_final(acc3, cnt1, z3)


# speedup vs baseline: 1.2547x; 1.2547x over previous
import jax
import jax.numpy as jnp
from jax import lax
from jax.experimental import pallas as pl
from jax.experimental.pallas import tpu as pltpu
from jax.experimental.pallas import tpu_sc as plsc

N = 10000
D = 128
E = 320000

NC = 2
NS = 16
NW = NC * NS
EPW = E // NW
CH = 100
NCH = EPW // CH
NP = 10240
RPS = NP // NS
CNTW = 16

_f32 = jnp.float32


def _build_segsum(with_count):
  mesh = plsc.VectorSubcoreMesh(
      core_axis_name="c", subcore_axis_name="s",
      num_cores=NC, num_subcores=NS)

  out_type = jax.ShapeDtypeStruct((NC, NP, D), _f32)
  scratch = [
      pltpu.MemorySpace.VMEM((NCH, CH), jnp.int32),
      pltpu.MemorySpace.VMEM((NCH, CH), jnp.int32),
      pltpu.MemorySpace.VMEM((CH, D), _f32),
      pltpu.MemorySpace.VMEM((CH, D), _f32),
      pltpu.MemorySpace.VMEM_SHARED((NP, D), _f32),
      pltpu.SemaphoreType.DMA,
      pltpu.SemaphoreType.DMA,
  ]

  def body(y, src, dst, zrow, acc_out,
           src_v, dst_v, rows0, rows1, acc_sh, sem0, sem1):
    c = lax.axis_index("c")
    s = lax.axis_index("s")
    wid = s * NC + c
    rows = (rows0, rows1)
    sems = (sem0, sem1)

    def gather(j, b):
      pltpu.async_copy(y.at[src_v.at[j]], rows[b], sems[b])

    pltpu.sync_copy(src.at[wid], src_v)
    pltpu.sync_copy(dst.at[wid], dst_v)
    gather(0, 0)
    gather(1, 1)
    pltpu.sync_copy(zrow, acc_sh.at[pl.ds(s * RPS, RPS)])
    plsc.subcore_barrier()

    def step(g, carry):
      j = 2 * g
      pltpu.make_async_copy(y.at[src_v.at[j]], rows0, sem0).wait()
      pltpu.sync_copy(rows0, acc_sh.at[dst_v.at[j]], add=True)
      gather(j + 2, 0)
      pltpu.make_async_copy(y.at[src_v.at[j + 1]], rows1, sem1).wait()
      pltpu.sync_copy(rows1, acc_sh.at[dst_v.at[j + 1]], add=True)
      gather(j + 3, 1)
      return carry

    lax.fori_loop(0, NCH // 2 - 1, step, 0)
    pltpu.make_async_copy(y.at[src_v.at[NCH - 2]], rows0, sem0).wait()
    pltpu.sync_copy(rows0, acc_sh.at[dst_v.at[NCH - 2]], add=True)
    pltpu.make_async_copy(y.at[src_v.at[NCH - 1]], rows1, sem1).wait()
    pltpu.sync_copy(rows1, acc_sh.at[dst_v.at[NCH - 1]], add=True)

    plsc.subcore_barrier()
    pltpu.sync_copy(acc_sh.at[pl.ds(s * RPS, RPS)],
                    acc_out.at[c, pl.ds(s * RPS, RPS)])

  return pl.kernel(
      body, out_type=out_type, mesh=mesh, scratch_types=scratch,
      compiler_params=pltpu.CompilerParams(use_tc_tiling_on_sc=False))


def _build_counts():
  mesh = plsc.VectorSubcoreMesh(
      core_axis_name="c", subcore_axis_name="s",
      num_cores=NC, num_subcores=NS)
  out_type = [jax.ShapeDtypeStruct((NC, NP, CNTW), _f32),
              jax.ShapeDtypeStruct((NC, NP, CNTW), _f32)]
  scratch = [
      pltpu.MemorySpace.VMEM((NCH, CH), jnp.int32),
      pltpu.MemorySpace.VMEM((CH, CNTW), _f32),
      pltpu.MemorySpace.VMEM_SHARED((NP, CNTW), _f32),
  ]

  def body(dst1, dst2, zcnt, ones, cnt1_out, cnt2_out,
           dst_v, ones_v, cnt_sh):
    c = lax.axis_index("c")
    s = lax.axis_index("s")
    wid = s * NC + c
    pltpu.sync_copy(ones, ones_v)
    for dst, cnt_out in ((dst1, cnt1_out), (dst2, cnt2_out)):
      pltpu.sync_copy(dst.at[wid], dst_v)
      pltpu.sync_copy(zcnt, cnt_sh.at[pl.ds(s * RPS, RPS)])
      plsc.subcore_barrier()

      def step(j, carry):
        pltpu.sync_copy(ones_v, cnt_sh.at[dst_v.at[j]], add=True)
        return carry

      lax.fori_loop(0, NCH, step, 0)
      plsc.subcore_barrier()
      pltpu.sync_copy(cnt_sh.at[pl.ds(s * RPS, RPS)],
                      cnt_out.at[c, pl.ds(s * RPS, RPS)])
      plsc.subcore_barrier()

  return pl.kernel(
      body, out_type=out_type, mesh=mesh, scratch_types=scratch,
      compiler_params=pltpu.CompilerParams(use_tc_tiling_on_sc=False))


_segsum = _build_segsum(True)
_counts = _build_counts()


R = 1000
G = N // R

_row_spec = pl.BlockSpec((R, D), lambda i: (i, 0))
_acc_spec = pl.BlockSpec((NC, R, D), lambda i: (0, i, 0))
_cnt_spec = pl.BlockSpec((NC, R, CNTW), lambda i: (0, i, 0))
_w_spec = pl.BlockSpec((D, D), lambda i: (0, 0))
_b_spec = pl.BlockSpec((1, D), lambda i: (0, 0))


def _tc_y1_body(x_ref, wlt_ref, y_ref):
  y_ref[...] = jnp.dot(x_ref[...], wlt_ref[...], preferred_element_type=_f32)


def _tc_z_body(h_ref, wrt_ref, bl_ref, z_ref):
  z_ref[...] = jnp.dot(h_ref[...], wrt_ref[...],
                       preferred_element_type=_f32) + bl_ref[...]


def _tc_comb_body(acc_ref, cnt_ref, z_ref, m_ref, wlt_ref, y_ref, h_ref):
  agg = acc_ref[0] + acc_ref[1]
  cnt = cnt_ref[0, :, 0:1] + cnt_ref[1, :, 0:1]
  inv = 1.0 / jnp.maximum(cnt, 1.0)
  h = jnp.maximum(z_ref[...] + agg * inv, 0.0) * m_ref[...]
  h_ref[...] = h
  y_ref[...] = jnp.dot(h, wlt_ref[...], preferred_element_type=_f32)


def _tc_final_body(acc_ref, cnt_ref, z_ref, out_ref):
  agg = acc_ref[0] + acc_ref[1]
  cnt = cnt_ref[0, :, 0:1] + cnt_ref[1, :, 0:1]
  inv = 1.0 / jnp.maximum(cnt, 1.0)
  out_ref[...] = z_ref[...] + agg * inv


_nd = jax.ShapeDtypeStruct((N, D), _f32)

_tc_y1 = pl.pallas_call(
    _tc_y1_body, grid=(G,),
    in_specs=[_row_spec, _w_spec],
    out_specs=_row_spec,
    out_shape=_nd)

_tc_z = pl.pallas_call(
    _tc_z_body, grid=(G,),
    in_specs=[_row_spec, _w_spec, _b_spec],
    out_specs=_row_spec,
    out_shape=_nd)

_tc_comb = pl.pallas_call(
    _tc_comb_body, grid=(G,),
    in_specs=[_acc_spec, _cnt_spec, _row_spec, _row_spec, _w_spec],
    out_specs=[_row_spec, _row_spec],
    out_shape=[_nd, _nd])

_tc_final = pl.pallas_call(
    _tc_final_body, grid=(G,),
    in_specs=[_acc_spec, _cnt_spec, _row_spec],
    out_specs=_row_spec,
    out_shape=_nd)


def kernel(x, edge_index, edge_idx_1_1, Wl1, bl1, Wr1, Wl2, bl2, Wr2,
           Wl3, bl3, Wr3):
  src1 = edge_index[0].reshape(NW, NCH, CH)
  dst1 = edge_index[1].reshape(NW, NCH, CH)
  src2 = edge_idx_1_1[0].reshape(NW, NCH, CH)
  dst2 = edge_idx_1_1[1].reshape(NW, NCH, CH)

  m1 = jax.random.bernoulli(jax.random.key(1), 0.5, (N, D)).astype(_f32) * 2.0
  m2 = jax.random.bernoulli(jax.random.key(2), 0.5, (N, D)).astype(_f32) * 2.0

  zrow = jnp.zeros((RPS, D), _f32)
  zcnt = jnp.zeros((RPS, CNTW), _f32)
  ones = jnp.ones((CH, CNTW), _f32)

  cnt1, cnt2 = _counts(dst1, dst2, zcnt, ones)
  y1 = _tc_y1(x, Wl1.T)
  acc1 = _segsum(y1, src1, dst1, zrow)
  z1 = _tc_z(x, Wr1.T, bl1.reshape(1, D))
  y2, h2 = _tc_comb(acc1, cnt1, z1, m1, Wl2.T)
  acc2 = _segsum(y2, src2, dst2, zrow)
  z2 = _tc_z(h2, Wr2.T, bl2.reshape(1, D))
  y3, h3 = _tc_comb(acc2, cnt2, z2, m2, Wl3.T)
  acc3 = _segsum(y3, src1, dst1, zrow)
  z3 = _tc_z(h3, Wr3.T, bl3.reshape(1, D))
  return _tc_final(acc3, cnt1, z3)


# speedup vs baseline: 1.2676x; 1.0103x over previous
import jax
import jax.numpy as jnp
from jax import lax
from jax.experimental import pallas as pl
from jax.experimental.pallas import tpu as pltpu
from jax.experimental.pallas import tpu_sc as plsc

N = 10000
D = 128
E = 320000

NC = 2
NS = 16
NW = NC * NS
EPW = E // NW
CH = 100
NCH = EPW // CH
NP = 10240
RPS = NP // NS
CNTW = 16

_f32 = jnp.float32


def _build_segsum():
  mesh = plsc.VectorSubcoreMesh(
      core_axis_name="c", subcore_axis_name="s",
      num_cores=NC, num_subcores=NS)

  out_type = jax.ShapeDtypeStruct((NC, NP, D), _f32)
  scratch = [
      pltpu.MemorySpace.VMEM((NCH, CH), jnp.int32),
      pltpu.MemorySpace.VMEM((NCH, CH), jnp.int32),
      pltpu.MemorySpace.VMEM((CH, D), _f32),
      pltpu.MemorySpace.VMEM((CH, D), _f32),
      pltpu.MemorySpace.VMEM_SHARED((NP, D), _f32),
      pltpu.SemaphoreType.DMA,
      pltpu.SemaphoreType.DMA,
  ]

  def body(y, src, dst, zrow, acc_out,
           src_v, dst_v, rows0, rows1, acc_sh, sem0, sem1):
    c = lax.axis_index("c")
    s = lax.axis_index("s")
    wid = s * NC + c
    rows = (rows0, rows1)
    sems = (sem0, sem1)

    def gather(j, b):
      pltpu.async_copy(y.at[src_v.at[j]], rows[b], sems[b])

    pltpu.sync_copy(src.at[wid], src_v)
    pltpu.sync_copy(dst.at[wid], dst_v)
    gather(0, 0)
    gather(1, 1)
    pltpu.sync_copy(zrow, acc_sh.at[pl.ds(s * RPS, RPS)])
    plsc.subcore_barrier()

    def step(g, carry):
      j = 2 * g
      pltpu.make_async_copy(y.at[src_v.at[j]], rows0, sem0).wait()
      pltpu.sync_copy(rows0, acc_sh.at[dst_v.at[j]], add=True)
      gather(j + 2, 0)
      pltpu.make_async_copy(y.at[src_v.at[j + 1]], rows1, sem1).wait()
      pltpu.sync_copy(rows1, acc_sh.at[dst_v.at[j + 1]], add=True)
      gather(j + 3, 1)
      return carry

    lax.fori_loop(0, NCH // 2 - 1, step, 0)
    pltpu.make_async_copy(y.at[src_v.at[NCH - 2]], rows0, sem0).wait()
    pltpu.sync_copy(rows0, acc_sh.at[dst_v.at[NCH - 2]], add=True)
    pltpu.make_async_copy(y.at[src_v.at[NCH - 1]], rows1, sem1).wait()
    pltpu.sync_copy(rows1, acc_sh.at[dst_v.at[NCH - 1]], add=True)

    plsc.subcore_barrier()
    pltpu.sync_copy(acc_sh.at[pl.ds(s * RPS, RPS)],
                    acc_out.at[c, pl.ds(s * RPS, RPS)])

  return pl.kernel(
      body, out_type=out_type, mesh=mesh, scratch_types=scratch,
      compiler_params=pltpu.CompilerParams(use_tc_tiling_on_sc=False))


def _build_counts():
  mesh = plsc.VectorSubcoreMesh(
      core_axis_name="c", subcore_axis_name="s",
      num_cores=NC, num_subcores=NS)
  out_type = [jax.ShapeDtypeStruct((NC, NP, CNTW), _f32),
              jax.ShapeDtypeStruct((NC, NP, CNTW), _f32)]
  scratch = [
      pltpu.MemorySpace.VMEM((NCH, CH), jnp.int32),
      pltpu.MemorySpace.VMEM((CH, CNTW), _f32),
      pltpu.MemorySpace.VMEM_SHARED((NP, CNTW), _f32),
  ]

  def body(dst1, dst2, zcnt, ones, cnt1_out, cnt2_out,
           dst_v, ones_v, cnt_sh):
    c = lax.axis_index("c")
    s = lax.axis_index("s")
    wid = s * NC + c
    pltpu.sync_copy(ones, ones_v)
    for dst, cnt_out in ((dst1, cnt1_out), (dst2, cnt2_out)):
      pltpu.sync_copy(dst.at[wid], dst_v)
      pltpu.sync_copy(zcnt, cnt_sh.at[pl.ds(s * RPS, RPS)])
      plsc.subcore_barrier()

      def step(j, carry):
        pltpu.sync_copy(ones_v, cnt_sh.at[dst_v.at[j]], add=True)
        return carry

      lax.fori_loop(0, NCH, step, 0)
      plsc.subcore_barrier()
      pltpu.sync_copy(cnt_sh.at[pl.ds(s * RPS, RPS)],
                      cnt_out.at[c, pl.ds(s * RPS, RPS)])
      plsc.subcore_barrier()

  return pl.kernel(
      body, out_type=out_type, mesh=mesh, scratch_types=scratch,
      compiler_params=pltpu.CompilerParams(use_tc_tiling_on_sc=False))


_segsum = _build_segsum()
_counts = _build_counts()


R = 1000
G = N // R

_row_spec = pl.BlockSpec((R, D), lambda i: (i, 0))
_acc_spec = pl.BlockSpec((NC, R, D), lambda i: (0, i, 0))
_cnt_spec = pl.BlockSpec((NC, R, CNTW), lambda i: (0, i, 0))
_w_spec = pl.BlockSpec((D, D), lambda i: (0, 0))
_b_spec = pl.BlockSpec((1, D), lambda i: (0, 0))


def _tc_y1_body(x_ref, wlt_ref, y_ref):
  y_ref[...] = jnp.dot(x_ref[...], wlt_ref[...], preferred_element_type=_f32)


def _tc_z_body(h_ref, wrt_ref, bl_ref, z_ref):
  z_ref[...] = jnp.dot(h_ref[...], wrt_ref[...],
                       preferred_element_type=_f32) + bl_ref[...]


def _tc_comb_body(acc_ref, cnt_ref, z_ref, m_ref, wlt_ref, y_ref, h_ref):
  agg = acc_ref[0] + acc_ref[1]
  cnt = cnt_ref[0, :, 0:1] + cnt_ref[1, :, 0:1]
  inv = 1.0 / jnp.maximum(cnt, 1.0)
  h = jnp.maximum(z_ref[...] + agg * inv, 0.0) * m_ref[...]
  h_ref[...] = h
  y_ref[...] = jnp.dot(h, wlt_ref[...], preferred_element_type=_f32)


def _tc_final_body(acc_ref, cnt_ref, z_ref, out_ref):
  agg = acc_ref[0] + acc_ref[1]
  cnt = cnt_ref[0, :, 0:1] + cnt_ref[1, :, 0:1]
  inv = 1.0 / jnp.maximum(cnt, 1.0)
  out_ref[...] = z_ref[...] + agg * inv


_nd = jax.ShapeDtypeStruct((N, D), _f32)

_tc_y1 = pl.pallas_call(
    _tc_y1_body, grid=(G,),
    in_specs=[_row_spec, _w_spec],
    out_specs=_row_spec,
    out_shape=_nd)

_tc_z = pl.pallas_call(
    _tc_z_body, grid=(G,),
    in_specs=[_row_spec, _w_spec, _b_spec],
    out_specs=_row_spec,
    out_shape=_nd)

_tc_comb = pl.pallas_call(
    _tc_comb_body, grid=(G,),
    in_specs=[_acc_spec, _cnt_spec, _row_spec, _row_spec, _w_spec],
    out_specs=[_row_spec, _row_spec],
    out_shape=[_nd, _nd])

_tc_final = pl.pallas_call(
    _tc_final_body, grid=(G,),
    in_specs=[_acc_spec, _cnt_spec, _row_spec],
    out_specs=_row_spec,
    out_shape=_nd)


def kernel(x, edge_index, edge_idx_1_1, Wl1, bl1, Wr1, Wl2, bl2, Wr2,
           Wl3, bl3, Wr3):
  src1 = edge_index[0].reshape(NW, NCH, CH)
  dst1 = edge_index[1].reshape(NW, NCH, CH)
  src2 = edge_idx_1_1[0].reshape(NW, NCH, CH)
  dst2 = edge_idx_1_1[1].reshape(NW, NCH, CH)

  m1 = jax.random.bernoulli(jax.random.key(1), 0.5, (N, D)).astype(_f32) * 2.0
  m2 = jax.random.bernoulli(jax.random.key(2), 0.5, (N, D)).astype(_f32) * 2.0

  zrow = jnp.zeros((RPS, D), _f32)
  zcnt = jnp.zeros((RPS, CNTW), _f32)
  ones = jnp.ones((CH, CNTW), _f32)

  cnt1, cnt2 = _counts(dst1, dst2, zcnt, ones)
  y1 = _tc_y1(x, Wl1.T)
  acc1 = _segsum(y1, src1, dst1, zrow)
  z1 = _tc_z(x, Wr1.T, bl1.reshape(1, D))
  y2, h2 = _tc_comb(acc1, cnt1, z1, m1, Wl2.T)
  acc2 = _segsum(y2, src2, dst2, zrow)
  z2 = _tc_z(h2, Wr2.T, bl2.reshape(1, D))
  y3, h3 = _tc_comb(acc2, cnt2, z2, m2, Wl3.T)
  acc3 = _segsum(y3, src1, dst1, zrow)
  z3 = _tc_z(h3, Wr3.T, bl3.reshape(1, D))
  return _tc_final(acc3, cnt1, z3)
